# baseline (speedup 1.0000x reference)
import functools

import jax
import jax.numpy as jnp
from jax import lax
from jax.experimental import pallas as pl
from jax.experimental.pallas import tpu as pltpu
from jax.experimental.pallas import tpu_sc as plsc

NC, NS = 2, 16
NW = NC * NS
B, L, EMB = 16384, 20, 32
LANES = 128
TOT = B * L
BPW = TOT // NW
ROWS_PW = B // NW
RB = 64
NRB = ROWS_PW // RB
C = 512
NCHUNK = BPW // C
NBUF = 3

_mesh = plsc.VectorSubcoreMesh(
    core_axis_name="c", subcore_axis_name="s", num_cores=NC, num_subcores=NS
)


@functools.partial(
    pl.kernel,
    mesh=_mesh,
    out_type=jax.ShapeDtypeStruct((TOT * EMB,), jnp.float32),
    scratch_types=[
        pltpu.VMEM((RB, LANES), jnp.int32),
        pltpu.VMEM((BPW + 16,), jnp.int32),
        pltpu.VMEM((NBUF, C, EMB), jnp.float32),
        pltpu.VMEM((NBUF, C * EMB), jnp.float32),
        pltpu.SemaphoreType.DMA((NBUF,)),
        pltpu.SemaphoreType.DMA((NBUF,)),
    ],
    compiler_params=pltpu.CompilerParams(
        use_tc_tiling_on_sc=False, needs_layout_passes=False
    ),
)
def _gather(y_hbm, table_hbm, out_hbm, yv, idx_v, rows_v, flat_v, gsem, ssem):
    wid = lax.axis_index("s") * NC + lax.axis_index("c")
    base = wid * BPW
    row_base = wid * ROWS_PW

    lane = lax.iota(jnp.int32, 16)

    def ext_block(b, carry):
        pltpu.sync_copy(y_hbm.at[pl.ds(row_base + b * RB, RB)], yv)

        def ext_row(r, c2):
            rvec = jnp.full((16,), r, jnp.int32)
            dst = (b * RB + r) * L
            lo = plsc.load_gather(yv, [rvec, lane])
            idx_v[pl.ds(dst, 16)] = lo
            hi = plsc.load_gather(yv, [rvec, lane + 16])
            idx_v[pl.ds(dst + 16, 16)] = hi
            return c2

        lax.fori_loop(0, RB, ext_row, 0)
        return carry

    lax.fori_loop(0, NRB, ext_block, 0)

    def fire_gather(c):
        return pltpu.async_copy(
            table_hbm.at[idx_v.at[pl.ds(c * C, C)]],
            rows_v.at[c % NBUF],
            gsem.at[c % NBUF],
        )

    def move_chunk(n):
        def mv(q, carry):
            for k in range(4):
                r = q * 4 + k
                o = q * 128 + k * 32
                flat_v[n, pl.ds(o, 16)] = rows_v[n, r, pl.ds(0, 16)]
                flat_v[n, pl.ds(o + 16, 16)] = rows_v[n, r, pl.ds(16, 16)]
            return carry

        lax.fori_loop(0, C // 4, mv, 0)

    def fire_store(c):
        return pltpu.async_copy(
            flat_v.at[c % NBUF],
            out_hbm.at[pl.ds((base + c * C) * EMB, C * EMB)],
            ssem.at[c % NBUF],
        )

    gathers = {}
    stores = {}
    for c in range(min(2, NCHUNK)):
        gathers[c] = fire_gather(c)
    for c in range(NCHUNK):
        if c + 2 < NCHUNK:
            gathers[c + 2] = fire_gather(c + 2)
        gathers.pop(c).wait()
        if c - NBUF >= 0:
            stores.pop(c - NBUF).wait()
        move_chunk(c % NBUF)
        stores[c] = fire_store(c)
    for c in sorted(stores):
        stores.pop(c).wait()


def kernel(y, table):
    y128 = jnp.pad(y, ((0, 0), (0, LANES - L)))
    out = _gather(y128, table)
    return out.reshape(B, L, EMB)


# speedup vs baseline: 1.0177x; 1.0177x over previous
import functools

import jax
import jax.numpy as jnp
from jax import lax
from jax.experimental import pallas as pl
from jax.experimental.pallas import tpu as pltpu
from jax.experimental.pallas import tpu_sc as plsc

NC, NS = 2, 16
NW = NC * NS
B, L, EMB = 16384, 20, 32
TOT = B * L
BPW = TOT // NW
C = 1024
NCHUNK = BPW // C
NBUF = 3

_mesh = plsc.VectorSubcoreMesh(
    core_axis_name="c", subcore_axis_name="s", num_cores=NC, num_subcores=NS
)


@functools.partial(
    pl.kernel,
    mesh=_mesh,
    out_type=jax.ShapeDtypeStruct((TOT, EMB), jnp.float32),
    scratch_types=[
        pltpu.VMEM((BPW,), jnp.int32),
        pltpu.VMEM((NBUF, C, EMB), jnp.float32),
        pltpu.SemaphoreType.DMA((NBUF,)),
        pltpu.SemaphoreType.DMA((NBUF,)),
    ],
    compiler_params=pltpu.CompilerParams(use_tc_tiling_on_sc=False),
)
def _gather(y_hbm, table_hbm, out_hbm, idx_v, rows_v, gsem, ssem):
    wid = lax.axis_index("s") * NC + lax.axis_index("c")
    base = wid * BPW

    pltpu.sync_copy(y_hbm.at[pl.ds(base, BPW)], idx_v)

    def fire_gather(c):
        return pltpu.async_copy(
            table_hbm.at[idx_v.at[pl.ds(c * C, C)]],
            rows_v.at[c % NBUF],
            gsem.at[c % NBUF],
        )

    def fire_store(c):
        return pltpu.async_copy(
            rows_v.at[c % NBUF],
            out_hbm.at[pl.ds(base + c * C, C)],
            ssem.at[c % NBUF],
        )

    gathers = {}
    stores = {}
    for c in range(min(2, NCHUNK)):
        gathers[c] = fire_gather(c)
    for c in range(NCHUNK):
        nxt = c + 2
        if nxt < NCHUNK:
            prev = nxt - NBUF
            if prev >= 0:
                stores.pop(prev).wait()
            gathers[nxt] = fire_gather(nxt)
        gathers.pop(c).wait()
        stores[c] = fire_store(c)
    for c in sorted(stores):
        stores.pop(c).wait()


def kernel(y, table):
    out = _gather(y.reshape(TOT), table)
    return out.reshape(B, L, EMB)
